# baseline (speedup 1.0000x reference)
import jax
import jax.numpy as jnp
from jax.experimental import pallas as pl


def _vq_body(z_ref, cb_ref, zsq_ref, csq_ref, zq_ref, idx_ref, lsum_ref):
    i = pl.program_id(0)
    zb = z_ref[...]
    cb = cb_ref[...]
    blk = zb.shape[0]
    ncodes = cb.shape[0]
    z_sq = zsq_ref[...]
    c_sq = csq_ref[...]
    zc = jax.lax.dot_general(
        zb, cb, (((1,), (1,)), ((), ())),
        preferred_element_type=jnp.float32)
    d2 = jnp.maximum((z_sq + c_sq) - 2.0 * zc, 0.0)
    dist = jnp.sqrt(d2)
    minval = jnp.min(dist, axis=1, keepdims=True)
    d2min = jnp.min(d2, axis=1)
    cols = jax.lax.broadcasted_iota(jnp.int32, (blk, ncodes), 1)
    idx = jnp.min(jnp.where(dist == minval, cols, ncodes), axis=1)
    idx_ref[0, 0, :] = idx
    onehot = (cols == idx[:, None]).astype(jnp.float32)
    zq_ref[...] = jax.lax.dot_general(
        onehot, cb, (((1,), (0,)), ((), ())),
        preferred_element_type=jnp.float32,
        precision=jax.lax.Precision.HIGHEST)

    part = jnp.sum(d2min).reshape(1, 1)

    @pl.when(i == 0)
    def _():
        lsum_ref[...] = jnp.zeros((1, 1), jnp.float32)

    lsum_ref[...] += part


def kernel(z, codebook):
    code_size = codebook.shape[1]
    ncodes = codebook.shape[0]
    n = z.shape[0] * z.shape[1]
    blk = 512
    nb = n // blk
    zf = z.reshape(n, code_size)
    z_sq = jnp.sum(zf * zf, axis=1, keepdims=True)
    c_sq = jnp.sum(codebook * codebook, axis=1)[None, :]
    zq, idx, lsum = pl.pallas_call(
        _vq_body,
        grid=(nb,),
        in_specs=[
            pl.BlockSpec((blk, code_size), lambda i: (i, 0)),
            pl.BlockSpec((ncodes, code_size), lambda i: (0, 0)),
            pl.BlockSpec((blk, 1), lambda i: (i, 0)),
            pl.BlockSpec((1, ncodes), lambda i: (0, 0)),
        ],
        out_specs=[
            pl.BlockSpec((blk, code_size), lambda i: (i, 0)),
            pl.BlockSpec((1, 1, blk), lambda i: (i, 0, 0)),
            pl.BlockSpec((1, 1), lambda i: (0, 0)),
        ],
        out_shape=[
            jax.ShapeDtypeStruct((n, code_size), jnp.float32),
            jax.ShapeDtypeStruct((nb, 1, blk), jnp.int32),
            jax.ShapeDtypeStruct((1, 1), jnp.float32),
        ],
    )(zf, codebook, z_sq, c_sq)
    loss = lsum[0, 0] / (n * code_size)
    return (zq.reshape(z.shape), loss, loss, idx.reshape(n, 1))


# speedup vs baseline: 1.1357x; 1.1357x over previous
import functools

import jax
import jax.numpy as jnp
from jax.experimental import pallas as pl
from jax.experimental.pallas import tpu as pltpu
from jax.experimental.pallas import tpu_sc as plsc


def _vq_body(z_ref, cb_ref, zsq_ref, csq_ref, idx_ref, lsum_ref):
    i = pl.program_id(0)
    zb = z_ref[...]
    cb = cb_ref[...]
    blk = zb.shape[0]
    ncodes = cb.shape[0]
    z_sq = zsq_ref[...]
    c_sq = csq_ref[...]
    zc = jax.lax.dot_general(
        zb, cb, (((1,), (1,)), ((), ())),
        preferred_element_type=jnp.float32)
    d2 = jnp.maximum((z_sq + c_sq) - 2.0 * zc, 0.0)
    dist = jnp.sqrt(d2)
    minval = jnp.min(dist, axis=1, keepdims=True)
    cols = jax.lax.broadcasted_iota(jnp.int32, (blk, ncodes), 1)
    idx = jnp.min(jnp.where(dist == minval, cols, ncodes), axis=1)
    idx_ref[0, 0, :] = idx
    part = jnp.sum(minval * minval).reshape(1, 1)

    @pl.when(i == 0)
    def _():
        lsum_ref[...] = jnp.zeros((1, 1), jnp.float32)

    lsum_ref[...] += part


def _make_sc_gather(ncodes, code_size, n):
    info = plsc.get_sparse_core_info()
    nc, ns = info.num_cores, info.num_subcores
    nw = nc * ns
    b_per_w = n // nw
    chunk = 96 if b_per_w % 96 == 0 else 72
    nchunk = b_per_w // chunk
    mesh = plsc.VectorSubcoreMesh(core_axis_name="c", subcore_axis_name="s")

    @functools.partial(
        pl.kernel, mesh=mesh,
        out_type=jax.ShapeDtypeStruct((n, code_size), jnp.float32),
        scratch_types=[
            pltpu.VMEM((b_per_w,), jnp.int32),
            pltpu.VMEM((b_per_w, code_size), jnp.float32),
            pltpu.SemaphoreType.DMA,
        ],
        compiler_params=pltpu.CompilerParams(use_tc_tiling_on_sc=False),
    )
    def gather(cb_hbm, idx_hbm, out_hbm, idx_v, rows_v, sem):
        wid = jax.lax.axis_index("s") * nc + jax.lax.axis_index("c")
        base = wid * b_per_w
        pltpu.sync_copy(idx_hbm.at[pl.ds(base, b_per_w)], idx_v)
        copies = [
            pltpu.async_copy(
                cb_hbm.at[idx_v.at[pl.ds(j * chunk, chunk)]],
                rows_v.at[pl.ds(j * chunk, chunk)], sem)
            for j in range(nchunk)
        ]
        for c in copies:
            c.wait()
        pltpu.sync_copy(rows_v, out_hbm.at[pl.ds(base, b_per_w)])

    return gather


def kernel(z, codebook):
    code_size = codebook.shape[1]
    ncodes = codebook.shape[0]
    n = z.shape[0] * z.shape[1]
    blk = 512
    nb = n // blk
    zf = z.reshape(n, code_size)
    z_sq = jnp.sum(zf * zf, axis=1, keepdims=True)
    c_sq = jnp.sum(codebook * codebook, axis=1)[None, :]
    idx, lsum = pl.pallas_call(
        _vq_body,
        grid=(nb,),
        in_specs=[
            pl.BlockSpec((blk, code_size), lambda i: (i, 0)),
            pl.BlockSpec((ncodes, code_size), lambda i: (0, 0)),
            pl.BlockSpec((blk, 1), lambda i: (i, 0)),
            pl.BlockSpec((1, ncodes), lambda i: (0, 0)),
        ],
        out_specs=[
            pl.BlockSpec((1, 1, blk), lambda i: (i, 0, 0)),
            pl.BlockSpec((1, 1), lambda i: (0, 0)),
        ],
        out_shape=[
            jax.ShapeDtypeStruct((nb, 1, blk), jnp.int32),
            jax.ShapeDtypeStruct((1, 1), jnp.float32),
        ],
    )(zf, codebook, z_sq, c_sq)
    idx_flat = idx.reshape(n)
    zq = _make_sc_gather(ncodes, code_size, n)(codebook, idx_flat)
    loss = lsum[0, 0] / (n * code_size)
    return (zq.reshape(z.shape), loss, loss, idx_flat.reshape(n, 1))


# speedup vs baseline: 1.3356x; 1.1761x over previous
import functools

import jax
import jax.numpy as jnp
from jax.experimental import pallas as pl
from jax.experimental.pallas import tpu as pltpu
from jax.experimental.pallas import tpu_sc as plsc


def _vq_body(z_ref, cb_ref, zsq_ref, csq_ref, idx_ref, lsum_ref):
    i = pl.program_id(0)
    zb2 = z_ref[...] * -2.0
    blk = zb2.shape[0]
    ncodes = cb_ref.shape[0]
    z_sq = zsq_ref[...]
    nch = 4
    cw = ncodes // nch
    minval, idx = None, None
    for ch in range(nch):
        cb = cb_ref[pl.ds(ch * cw, cw), :]
        c_sq = csq_ref[pl.ds(ch * cw, cw), :]
        zc = jax.lax.dot_general(
            cb, zb2, (((1,), (1,)), ((), ())),
            preferred_element_type=jnp.float32)
        d2 = jnp.maximum((z_sq + c_sq) + zc, 0.0)
        dist = jnp.sqrt(d2)
        mv = jnp.min(dist, axis=0)
        rows = jax.lax.broadcasted_iota(jnp.int32, (cw, blk), 0)
        ii = jnp.min(jnp.where(dist == mv[None, :], rows, cw), axis=0)
        ii = ii + ch * cw
        if minval is None:
            minval, idx = mv, ii
        else:
            idx = jnp.where(mv < minval, ii, idx)
            minval = jnp.minimum(minval, mv)
    idx_ref[0, 0, :] = idx
    part = jnp.sum(minval * minval).reshape(1, 1)

    @pl.when(i == 0)
    def _():
        lsum_ref[...] = jnp.zeros((1, 1), jnp.float32)

    lsum_ref[...] += part


def _make_sc_gather(ncodes, code_size, n):
    info = plsc.get_sparse_core_info()
    nc, ns = info.num_cores, info.num_subcores
    nw = nc * ns
    b_per_w = n // nw
    chunk = 96 if b_per_w % 96 == 0 else 72
    nchunk = b_per_w // chunk
    mesh = plsc.VectorSubcoreMesh(core_axis_name="c", subcore_axis_name="s")

    @functools.partial(
        pl.kernel, mesh=mesh,
        out_type=jax.ShapeDtypeStruct((n, code_size), jnp.float32),
        scratch_types=[
            pltpu.VMEM((b_per_w,), jnp.int32),
            pltpu.VMEM((b_per_w, code_size), jnp.float32),
            pltpu.SemaphoreType.DMA,
        ],
        compiler_params=pltpu.CompilerParams(use_tc_tiling_on_sc=False),
    )
    def gather(cb_hbm, idx_hbm, out_hbm, idx_v, rows_v, sem):
        wid = jax.lax.axis_index("s") * nc + jax.lax.axis_index("c")
        base = wid * b_per_w
        pltpu.sync_copy(idx_hbm.at[pl.ds(base, b_per_w)], idx_v)
        copies = [
            pltpu.async_copy(
                cb_hbm.at[idx_v.at[pl.ds(j * chunk, chunk)]],
                rows_v.at[pl.ds(j * chunk, chunk)], sem)
            for j in range(nchunk)
        ]
        for c in copies:
            c.wait()
        pltpu.sync_copy(rows_v, out_hbm.at[pl.ds(base, b_per_w)])

    return gather


def kernel(z, codebook):
    code_size = codebook.shape[1]
    ncodes = codebook.shape[0]
    n = z.shape[0] * z.shape[1]
    blk = 512
    nb = n // blk
    zf = z.reshape(n, code_size)
    z_sq = jnp.sum(zf * zf, axis=1)[None, :]
    c_sq = jnp.sum(codebook * codebook, axis=1)[:, None]
    idx, lsum = pl.pallas_call(
        _vq_body,
        grid=(nb,),
        in_specs=[
            pl.BlockSpec((blk, code_size), lambda i: (i, 0)),
            pl.BlockSpec((ncodes, code_size), lambda i: (0, 0)),
            pl.BlockSpec((1, blk), lambda i: (0, i)),
            pl.BlockSpec((ncodes, 1), lambda i: (0, 0)),
        ],
        out_specs=[
            pl.BlockSpec((1, 1, blk), lambda i: (i, 0, 0)),
            pl.BlockSpec((1, 1), lambda i: (0, 0)),
        ],
        out_shape=[
            jax.ShapeDtypeStruct((nb, 1, blk), jnp.int32),
            jax.ShapeDtypeStruct((1, 1), jnp.float32),
        ],
    )(zf, codebook, z_sq, c_sq)
    idx_flat = idx.reshape(n)
    zq = _make_sc_gather(ncodes, code_size, n)(codebook, idx_flat)
    loss = lsum[0, 0] / (n * code_size)
    return (zq.reshape(z.shape), loss, loss, idx_flat.reshape(n, 1))


# speedup vs baseline: 1.4208x; 1.0638x over previous
import functools

import jax
import jax.numpy as jnp
from jax.experimental import pallas as pl
from jax.experimental.pallas import tpu as pltpu
from jax.experimental.pallas import tpu_sc as plsc


def _vq_body(z_ref, cb_ref, zsq_ref, csq_ref, idx_ref, lsum_ref):
    i = pl.program_id(0)
    zb2 = z_ref[...] * -2.0
    blk = zb2.shape[0]
    ncodes = cb_ref.shape[0]
    z_sq = zsq_ref[...]
    nch = 4
    cw = ncodes // nch
    minval, idx = None, None
    for ch in range(nch):
        cb = cb_ref[pl.ds(ch * cw, cw), :]
        c_sq = csq_ref[pl.ds(ch * cw, cw), :]
        zc = jax.lax.dot_general(
            cb, zb2, (((1,), (1,)), ((), ())),
            preferred_element_type=jnp.float32)
        d2 = jnp.maximum((z_sq + c_sq) + zc, 0.0)
        dist = jnp.where(d2 > 0.0, d2 * jax.lax.rsqrt(d2), 0.0)
        mv = jnp.min(dist, axis=0)
        rows = jax.lax.broadcasted_iota(jnp.int32, (cw, blk), 0)
        ii = jnp.min(jnp.where(dist == mv[None, :], rows, cw), axis=0)
        ii = ii + ch * cw
        if minval is None:
            minval, idx = mv, ii
        else:
            idx = jnp.where(mv < minval, ii, idx)
            minval = jnp.minimum(minval, mv)
    idx_ref[0, 0, :] = idx
    part = jnp.sum(minval * minval).reshape(1, 1)

    @pl.when(i == 0)
    def _():
        lsum_ref[...] = jnp.zeros((1, 1), jnp.float32)

    lsum_ref[...] += part

    nelem = pl.num_programs(0) * blk * zb2.shape[1]

    @pl.when(i == pl.num_programs(0) - 1)
    def _():
        lsum_ref[...] = lsum_ref[...] / float(nelem)


def _make_sc_gather(ncodes, code_size, n):
    info = plsc.get_sparse_core_info()
    nc, ns = info.num_cores, info.num_subcores
    nw = nc * ns
    b_per_w = n // nw
    chunk = 96 if b_per_w % 96 == 0 else 72
    nchunk = b_per_w // chunk
    mesh = plsc.VectorSubcoreMesh(core_axis_name="c", subcore_axis_name="s")

    @functools.partial(
        pl.kernel, mesh=mesh,
        out_type=jax.ShapeDtypeStruct((n, code_size), jnp.float32),
        scratch_types=[
            pltpu.VMEM((b_per_w,), jnp.int32),
            pltpu.VMEM((b_per_w, code_size), jnp.float32),
            pltpu.SemaphoreType.DMA,
        ],
        compiler_params=pltpu.CompilerParams(use_tc_tiling_on_sc=False),
    )
    def gather(cb_hbm, idx_hbm, out_hbm, idx_v, rows_v, sem):
        wid = jax.lax.axis_index("s") * nc + jax.lax.axis_index("c")
        base = wid * b_per_w
        pltpu.sync_copy(idx_hbm.at[pl.ds(base, b_per_w)], idx_v)
        copies = [
            pltpu.async_copy(
                cb_hbm.at[idx_v.at[pl.ds(j * chunk, chunk)]],
                rows_v.at[pl.ds(j * chunk, chunk)], sem)
            for j in range(nchunk)
        ]
        for c in copies:
            c.wait()
        pltpu.sync_copy(rows_v, out_hbm.at[pl.ds(base, b_per_w)])

    return gather


def kernel(z, codebook):
    code_size = codebook.shape[1]
    ncodes = codebook.shape[0]
    n = z.shape[0] * z.shape[1]
    blk = 512
    nb = n // blk
    zf = z.reshape(n, code_size)
    z_sq = jnp.sum(zf * zf, axis=1)[None, :]
    c_sq = jnp.sum(codebook * codebook, axis=1)[:, None]
    idx, lsum = pl.pallas_call(
        _vq_body,
        grid=(nb,),
        in_specs=[
            pl.BlockSpec((blk, code_size), lambda i: (i, 0)),
            pl.BlockSpec((ncodes, code_size), lambda i: (0, 0)),
            pl.BlockSpec((1, blk), lambda i: (0, i)),
            pl.BlockSpec((ncodes, 1), lambda i: (0, 0)),
        ],
        out_specs=[
            pl.BlockSpec((1, 1, blk), lambda i: (i, 0, 0)),
            pl.BlockSpec((1, 1), lambda i: (0, 0)),
        ],
        out_shape=[
            jax.ShapeDtypeStruct((nb, 1, blk), jnp.int32),
            jax.ShapeDtypeStruct((1, 1), jnp.float32),
        ],
    )(zf, codebook, z_sq, c_sq)
    idx_flat = idx.reshape(n)
    zq = _make_sc_gather(ncodes, code_size, n)(codebook, idx_flat)
    loss = lsum.reshape(())
    return (zq.reshape(z.shape), loss, loss, idx_flat.reshape(n, 1))


# speedup vs baseline: 1.4689x; 1.0338x over previous
import functools

import jax
import jax.numpy as jnp
from jax.experimental import pallas as pl
from jax.experimental.pallas import tpu as pltpu
from jax.experimental.pallas import tpu_sc as plsc


def _vq_body(z_ref, cb_ref, zsq_ref, csq_ref, idx_ref, lsum_ref):
    i = pl.program_id(0)
    zb2 = z_ref[...] * -2.0
    blk = zb2.shape[0]
    ncodes = cb_ref.shape[0]
    z_sq = zsq_ref[...]
    nch = 4
    cw = ncodes // nch
    minval, idx = None, None
    for ch in range(nch):
        cb = cb_ref[pl.ds(ch * cw, cw), :]
        c_sq = csq_ref[pl.ds(ch * cw, cw), :]
        zc = jax.lax.dot_general(
            cb, zb2, (((1,), (1,)), ((), ())),
            preferred_element_type=jnp.float32)
        d2 = jnp.maximum((z_sq + c_sq) + zc, 0.0)
        dist = jnp.where(d2 > 0.0, d2 * jax.lax.rsqrt(d2), 0.0)
        mv = jnp.min(dist, axis=0)
        rows = jax.lax.broadcasted_iota(jnp.int32, (cw, blk), 0)
        ii = jnp.min(jnp.where(dist == mv[None, :], rows, cw), axis=0)
        ii = ii + ch * cw
        if minval is None:
            minval, idx = mv, ii
        else:
            idx = jnp.where(mv < minval, ii, idx)
            minval = jnp.minimum(minval, mv)
    idx_ref[0, 0, :] = idx
    part = jnp.sum(minval * minval).reshape(1, 1)

    @pl.when(i == 0)
    def _():
        lsum_ref[...] = jnp.zeros((1, 1), jnp.float32)

    lsum_ref[...] += part

    nelem = pl.num_programs(0) * blk * zb2.shape[1]

    @pl.when(i == pl.num_programs(0) - 1)
    def _():
        lsum_ref[...] = lsum_ref[...] / float(nelem)


def _make_sc_gather(ncodes, code_size, n):
    info = plsc.get_sparse_core_info()
    nc, ns = info.num_cores, info.num_subcores
    nw = nc * ns
    b_per_w = n // nw
    chunk = 96 if b_per_w % 96 == 0 else 72
    nchunk = b_per_w // chunk
    mesh = plsc.VectorSubcoreMesh(core_axis_name="c", subcore_axis_name="s")

    @functools.partial(
        pl.kernel, mesh=mesh,
        out_type=jax.ShapeDtypeStruct((n, code_size), jnp.float32),
        scratch_types=[
            pltpu.VMEM((b_per_w,), jnp.int32),
            pltpu.VMEM((b_per_w, code_size), jnp.float32),
            pltpu.SemaphoreType.DMA,
        ],
        compiler_params=pltpu.CompilerParams(use_tc_tiling_on_sc=False),
    )
    def gather(cb_hbm, idx_hbm, out_hbm, idx_v, rows_v, sem):
        wid = jax.lax.axis_index("s") * nc + jax.lax.axis_index("c")
        base = wid * b_per_w
        pltpu.sync_copy(idx_hbm.at[pl.ds(base, b_per_w)], idx_v)
        copies = [
            pltpu.async_copy(
                cb_hbm.at[idx_v.at[pl.ds(j * chunk, chunk)]],
                rows_v.at[pl.ds(j * chunk, chunk)], sem)
            for j in range(nchunk)
        ]
        for c in copies:
            c.wait()
        pltpu.sync_copy(rows_v, out_hbm.at[pl.ds(base, b_per_w)])

    return gather


def kernel(z, codebook):
    code_size = codebook.shape[1]
    ncodes = codebook.shape[0]
    n = z.shape[0] * z.shape[1]
    blk = 1024
    nb = n // blk
    zf = z.reshape(n, code_size)
    z_sq = jnp.sum(zf * zf, axis=1)[None, :]
    c_sq = jnp.sum(codebook * codebook, axis=1)[:, None]
    idx, lsum = pl.pallas_call(
        _vq_body,
        grid=(nb,),
        in_specs=[
            pl.BlockSpec((blk, code_size), lambda i: (i, 0)),
            pl.BlockSpec((ncodes, code_size), lambda i: (0, 0)),
            pl.BlockSpec((1, blk), lambda i: (0, i)),
            pl.BlockSpec((ncodes, 1), lambda i: (0, 0)),
        ],
        out_specs=[
            pl.BlockSpec((1, 1, blk), lambda i: (i, 0, 0)),
            pl.BlockSpec((1, 1), lambda i: (0, 0)),
        ],
        out_shape=[
            jax.ShapeDtypeStruct((nb, 1, blk), jnp.int32),
            jax.ShapeDtypeStruct((1, 1), jnp.float32),
        ],
    )(zf, codebook, z_sq, c_sq)
    idx_flat = idx.reshape(n)
    zq = _make_sc_gather(ncodes, code_size, n)(codebook, idx_flat)
    loss = lsum.reshape(())
    return (zq.reshape(z.shape), loss, loss, idx_flat.reshape(n, 1))


# speedup vs baseline: 1.5150x; 1.0314x over previous
import functools

import jax
import jax.numpy as jnp
from jax.experimental import pallas as pl
from jax.experimental.pallas import tpu as pltpu
from jax.experimental.pallas import tpu_sc as plsc


def _vq_body(z_ref, cb_ref, zsq_ref, csq_ref, idx_ref, lsum_ref):
    i = pl.program_id(0)
    zb2 = z_ref[...] * -2.0
    blk = zb2.shape[0]
    ncodes = cb_ref.shape[0]
    z_sq = zsq_ref[...]
    nch = 4
    cw = ncodes // nch
    minval, idx = None, None
    for ch in range(nch):
        cb = cb_ref[pl.ds(ch * cw, cw), :]
        c_sq = csq_ref[pl.ds(ch * cw, cw), :]
        zc = jax.lax.dot_general(
            cb, zb2, (((1,), (1,)), ((), ())),
            preferred_element_type=jnp.float32)
        d2 = jnp.maximum((z_sq + c_sq) + zc, 0.0)
        dist = jnp.where(d2 > 0.0, d2 * jax.lax.rsqrt(d2), 0.0)
        mv = jnp.min(dist, axis=0)
        rows = jax.lax.broadcasted_iota(jnp.int32, (cw, blk), 0)
        ii = jnp.min(jnp.where(dist == mv[None, :], rows, cw), axis=0)
        ii = ii + ch * cw
        if minval is None:
            minval, idx = mv, ii
        else:
            idx = jnp.where(mv < minval, ii, idx)
            minval = jnp.minimum(minval, mv)
    idx_ref[0, 0, :] = idx
    part = jnp.sum(minval * minval).reshape(1, 1)

    @pl.when(i == 0)
    def _():
        lsum_ref[...] = jnp.zeros((1, 1), jnp.float32)

    lsum_ref[...] += part

    nelem = pl.num_programs(0) * blk * zb2.shape[1]

    @pl.when(i == pl.num_programs(0) - 1)
    def _():
        lsum_ref[...] = lsum_ref[...] / float(nelem)


def _make_sc_gather(ncodes, code_size, n):
    info = plsc.get_sparse_core_info()
    nc, ns = info.num_cores, info.num_subcores
    nw = nc * ns
    b_per_w = n // nw
    chunk = 96 if b_per_w % 96 == 0 else 72
    nchunk = b_per_w // chunk
    mesh = plsc.VectorSubcoreMesh(core_axis_name="c", subcore_axis_name="s")

    @functools.partial(
        pl.kernel, mesh=mesh,
        out_type=jax.ShapeDtypeStruct((n, code_size), jnp.float32),
        scratch_types=[
            pltpu.VMEM((b_per_w,), jnp.int32),
            pltpu.VMEM((b_per_w, code_size), jnp.float32),
            pltpu.SemaphoreType.DMA,
        ],
        compiler_params=pltpu.CompilerParams(use_tc_tiling_on_sc=False),
    )
    def gather(cb_hbm, idx_hbm, out_hbm, idx_v, rows_v, sem):
        wid = jax.lax.axis_index("s") * nc + jax.lax.axis_index("c")
        base = wid * b_per_w
        pltpu.sync_copy(idx_hbm.at[pl.ds(base, b_per_w)], idx_v)
        copies = [
            pltpu.async_copy(
                cb_hbm.at[idx_v.at[pl.ds(j * chunk, chunk)]],
                rows_v.at[pl.ds(j * chunk, chunk)], sem)
            for j in range(nchunk)
        ]
        for c in copies:
            c.wait()
        pltpu.sync_copy(rows_v, out_hbm.at[pl.ds(base, b_per_w)])

    return gather


def kernel(z, codebook):
    code_size = codebook.shape[1]
    ncodes = codebook.shape[0]
    n = z.shape[0] * z.shape[1]
    blk = 2304
    nb = n // blk
    zf = z.reshape(n, code_size)
    z_sq = jnp.sum(zf * zf, axis=1)[None, :]
    c_sq = jnp.sum(codebook * codebook, axis=1)[:, None]
    idx, lsum = pl.pallas_call(
        _vq_body,
        grid=(nb,),
        in_specs=[
            pl.BlockSpec((blk, code_size), lambda i: (i, 0)),
            pl.BlockSpec((ncodes, code_size), lambda i: (0, 0)),
            pl.BlockSpec((1, blk), lambda i: (0, i)),
            pl.BlockSpec((ncodes, 1), lambda i: (0, 0)),
        ],
        out_specs=[
            pl.BlockSpec((1, 1, blk), lambda i: (i, 0, 0)),
            pl.BlockSpec((1, 1), lambda i: (0, 0)),
        ],
        out_shape=[
            jax.ShapeDtypeStruct((nb, 1, blk), jnp.int32),
            jax.ShapeDtypeStruct((1, 1), jnp.float32),
        ],
    )(zf, codebook, z_sq, c_sq)
    idx_flat = idx.reshape(n)
    zq = _make_sc_gather(ncodes, code_size, n)(codebook, idx_flat)
    loss = lsum.reshape(())
    return (zq.reshape(z.shape), loss, loss, idx_flat.reshape(n, 1))
